# baseline (speedup 1.0000x reference)
import functools

import jax
import jax.numpy as jnp
from jax import lax
from jax.experimental import pallas as pl
from jax.experimental.pallas import tpu as pltpu
from jax.experimental.pallas import tpu_sc as plsc

N_NODES = 10000
N_EDGES = 320000
EMB = 128

NC = 2
NS = 16
NW = NC * NS

EPT = N_EDGES // NW
K = 100
NCHUNK = EPT // K
WB = 80
N_PAD = 10240
RPT = N_PAD // NS


def _mlp_body(x_ref, w1_ref, b1_ref, w2_ref, b2_ref, o_ref):
    h = jnp.dot(x_ref[...], w1_ref[...], preferred_element_type=jnp.float32)
    h = jnp.maximum(h + b1_ref[...], 0.0)
    h = jnp.dot(h, w2_ref[...], preferred_element_type=jnp.float32)
    o_ref[...] = jnp.maximum(h + b2_ref[...], 0.0)


def _mlp(X, W1, b1, W2, b2):
    blk = 1000
    grid = (N_NODES // blk,)
    return pl.pallas_call(
        _mlp_body,
        grid=grid,
        in_specs=[
            pl.BlockSpec((blk, EMB), lambda i: (i, 0)),
            pl.BlockSpec((EMB, EMB), lambda i: (0, 0)),
            pl.BlockSpec((1, EMB), lambda i: (0, 0)),
            pl.BlockSpec((EMB, EMB), lambda i: (0, 0)),
            pl.BlockSpec((1, EMB), lambda i: (0, 0)),
        ],
        out_specs=pl.BlockSpec((blk, EMB), lambda i: (i, 0)),
        out_shape=jax.ShapeDtypeStruct((N_NODES, EMB), jnp.float32),
    )(X, W1, b1, W2, b2)


@functools.partial(
    pl.kernel,
    mesh=plsc.VectorSubcoreMesh(core_axis_name="c", subcore_axis_name="s"),
    out_type=jax.ShapeDtypeStruct((NC, N_PAD, EMB), jnp.float32),
    scratch_types=[
        pltpu.VMEM((2, 2, K), jnp.int32),
        pltpu.VMEM((2, 2, K), jnp.int32),
        pltpu.VMEM((K, EMB), jnp.float32),
        pltpu.VMEM((K, EMB), jnp.float32),
        pltpu.VMEM_SHARED((N_PAD, EMB), jnp.float32),
        pltpu.SemaphoreType.DMA,
        pltpu.SemaphoreType.DMA,
        pltpu.SemaphoreType.DMA,
        pltpu.SemaphoreType.DMA,
        pltpu.SemaphoreType.DMA,
        pltpu.SemaphoreType.DMA,
    ],
)
def _scatter_kernel(h_hbm, idx_hbm, out_hbm,
                    pxA, pxB, rows0, rows1, acc, siA, siB, sg0, sg1, ss0, ss1):
    c = lax.axis_index("c")
    s = lax.axis_index("s")
    wid = c * NS + s

    zero16 = jnp.zeros((16,), jnp.float32)

    def zero_row(r, carry):
        for j in range(EMB // 16):
            rows0[r, pl.ds(j * 16, 16)] = zero16
        return carry

    lax.fori_loop(0, WB, zero_row, 0)
    zsrc = rows0.at[pl.ds(0, WB)]
    for j in range(RPT // WB):
        pltpu.sync_copy(zsrc, acc.at[pl.ds(s * RPT + j * WB, WB)])
    plsc.subcore_barrier()

    last_pair = NCHUNK - 2

    def start_pair(i, px, sem):
        pltpu.async_copy(idx_hbm.at[wid, pl.ds(i, 2)], px, sem)

    def wait_pair(px, sem):
        pltpu.make_async_copy(idx_hbm.at[wid, pl.ds(0, 2)], px, sem).wait()

    def start_gather(isrc, buf, sem):
        pltpu.async_copy(h_hbm.at[isrc], buf, sem)

    def wait_gather(isrc, buf, sem):
        pltpu.make_async_copy(h_hbm.at[isrc], buf, sem).wait()

    def start_scatter(idst, buf, sem):
        pltpu.async_copy(buf, acc.at[idst], sem, add=True)

    def wait_scatter(idst, buf, sem):
        pltpu.make_async_copy(buf, acc.at[idst], sem).wait()

    start_pair(0, pxA, siA)
    start_pair(2, pxB, siB)
    wait_pair(pxA, siA)
    start_gather(pxA.at[0, 0], rows0, sg0)
    start_gather(pxA.at[1, 0], rows1, sg1)

    def quad(g, carry):
        q = 4 * g
        wait_gather(pxA.at[0, 0], rows0, sg0)
        start_scatter(pxA.at[0, 1], rows0, ss0)
        wait_gather(pxA.at[1, 0], rows1, sg1)
        start_scatter(pxA.at[1, 1], rows1, ss1)
        wait_pair(pxB, siB)
        wait_scatter(pxA.at[0, 1], rows0, ss0)
        start_gather(pxB.at[0, 0], rows0, sg0)
        wait_scatter(pxA.at[1, 1], rows1, ss1)
        start_gather(pxB.at[1, 0], rows1, sg1)
        start_pair(jnp.minimum(q + 4, last_pair), pxA, siA)
        wait_gather(pxB.at[0, 0], rows0, sg0)
        start_scatter(pxB.at[0, 1], rows0, ss0)
        wait_gather(pxB.at[1, 0], rows1, sg1)
        start_scatter(pxB.at[1, 1], rows1, ss1)
        wait_pair(pxA, siA)
        wait_scatter(pxB.at[0, 1], rows0, ss0)
        start_gather(pxA.at[0, 0], rows0, sg0)
        wait_scatter(pxB.at[1, 1], rows1, ss1)
        start_gather(pxA.at[1, 0], rows1, sg1)
        start_pair(jnp.minimum(q + 6, last_pair), pxB, siB)
        return carry

    lax.fori_loop(0, NCHUNK // 4, quad, 0)
    wait_gather(pxA.at[0, 0], rows0, sg0)
    wait_gather(pxA.at[1, 0], rows1, sg1)
    wait_pair(pxB, siB)
    plsc.subcore_barrier()

    for j in range(RPT // WB):
        r0 = s * RPT + j * WB
        pltpu.sync_copy(acc.at[pl.ds(r0, WB)], zsrc)
        pltpu.sync_copy(zsrc, out_hbm.at[c, pl.ds(r0, WB)])


def _add_body(p_ref, o_ref):
    o_ref[...] = p_ref[0] + p_ref[1]


def _combine(partials):
    blk = 1000
    grid = (N_NODES // blk,)
    return pl.pallas_call(
        _add_body,
        grid=grid,
        in_specs=[pl.BlockSpec((NC, blk, EMB), lambda i: (0, i, 0))],
        out_specs=pl.BlockSpec((blk, EMB), lambda i: (i, 0)),
        out_shape=jax.ShapeDtypeStruct((N_NODES, EMB), jnp.float32),
    )(partials)


def kernel(X, edge_index, W1, b1, W2, b2):
    h = _mlp(X, W1, b1.reshape(1, EMB), W2, b2.reshape(1, EMB))
    idx = jnp.stack(
        [edge_index[0].astype(jnp.int32).reshape(NW, NCHUNK, K),
         edge_index[1].astype(jnp.int32).reshape(NW, NCHUNK, K)], axis=2)
    partials = _scatter_kernel(h, idx)
    return _combine(partials)


# speedup vs baseline: 1.3725x; 1.3725x over previous
import functools

import jax
import jax.numpy as jnp
from jax import lax
from jax.experimental import pallas as pl
from jax.experimental.pallas import tpu as pltpu
from jax.experimental.pallas import tpu_sc as plsc

N_NODES = 10000
N_EDGES = 320000
EMB = 128

NC = 2
NS = 16
NW = NC * NS

EPT = N_EDGES // NW
K = 100
NCHUNK = EPT // K
WB = 80
N_PAD = 10240
RPT = N_PAD // NS


def _mlp_body(x_ref, w1_ref, b1_ref, w2_ref, b2_ref, o_ref):
    h = jnp.dot(x_ref[...], w1_ref[...], preferred_element_type=jnp.float32)
    h = jnp.maximum(h + b1_ref[...], 0.0)
    h = jnp.dot(h, w2_ref[...], preferred_element_type=jnp.float32)
    o_ref[...] = jnp.maximum(h + b2_ref[...], 0.0)


def _mlp(X, W1, b1, W2, b2):
    blk = 1000
    grid = (N_NODES // blk,)
    return pl.pallas_call(
        _mlp_body,
        grid=grid,
        in_specs=[
            pl.BlockSpec((blk, EMB), lambda i: (i, 0)),
            pl.BlockSpec((EMB, EMB), lambda i: (0, 0)),
            pl.BlockSpec((1, EMB), lambda i: (0, 0)),
            pl.BlockSpec((EMB, EMB), lambda i: (0, 0)),
            pl.BlockSpec((1, EMB), lambda i: (0, 0)),
        ],
        out_specs=pl.BlockSpec((blk, EMB), lambda i: (i, 0)),
        out_shape=jax.ShapeDtypeStruct((N_NODES, EMB), jnp.float32),
    )(X, W1, b1, W2, b2)


@functools.partial(
    pl.kernel,
    mesh=plsc.VectorSubcoreMesh(core_axis_name="c", subcore_axis_name="s"),
    out_type=jax.ShapeDtypeStruct((NC, N_PAD, EMB), jnp.float32),
    scratch_types=[
        pltpu.VMEM((2, K), jnp.int32),
        pltpu.VMEM((2, K), jnp.int32),
        pltpu.VMEM((2, K), jnp.int32),
        pltpu.VMEM((2, K), jnp.int32),
        pltpu.VMEM((K, EMB), jnp.float32),
        pltpu.VMEM((K, EMB), jnp.float32),
        pltpu.VMEM_SHARED((N_PAD, EMB), jnp.float32),
        pltpu.SemaphoreType.DMA,
        pltpu.SemaphoreType.DMA,
        pltpu.SemaphoreType.DMA,
        pltpu.SemaphoreType.DMA,
    ],
)
def _scatter_kernel(h_hbm, idx_hbm, out_hbm,
                    sxA, dxA, sxB, dxB, rows0, rows1, acc, siA, siB, sg0, sg1):
    c = lax.axis_index("c")
    s = lax.axis_index("s")
    wid = c * NS + s

    zero16 = jnp.zeros((16,), jnp.float32)

    def zero_row(r, carry):
        for j in range(EMB // 16):
            rows0[r, pl.ds(j * 16, 16)] = zero16
        return carry

    lax.fori_loop(0, WB, zero_row, 0)
    zsrc = rows0.at[pl.ds(0, WB)]
    for j in range(RPT // WB):
        pltpu.sync_copy(zsrc, acc.at[pl.ds(s * RPT + j * WB, WB)])
    plsc.subcore_barrier()

    last_pair = NCHUNK - 2

    def start_pair(i, sx, dx, sem):
        pltpu.async_copy(idx_hbm.at[0, wid, pl.ds(i, 2)], sx, sem)
        pltpu.async_copy(idx_hbm.at[1, wid, pl.ds(i, 2)], dx, sem)

    def wait_pair(sx, dx, sem):
        pltpu.make_async_copy(idx_hbm.at[0, wid, pl.ds(0, 2)], sx, sem).wait()
        pltpu.make_async_copy(idx_hbm.at[1, wid, pl.ds(0, 2)], dx, sem).wait()

    def start_gather(isrc, buf, sem):
        pltpu.async_copy(h_hbm.at[isrc], buf, sem)

    def wait_gather(isrc, buf, sem):
        pltpu.make_async_copy(h_hbm.at[isrc], buf, sem).wait()

    def scatter(idst, buf):
        pltpu.sync_copy(buf, acc.at[idst], add=True)

    start_pair(0, sxA, dxA, siA)
    start_pair(2, sxB, dxB, siB)
    wait_pair(sxA, dxA, siA)
    start_gather(sxA.at[0], rows0, sg0)
    start_gather(sxA.at[1], rows1, sg1)

    def quad(g, carry):
        q = 4 * g
        wait_pair(sxB, dxB, siB)
        wait_gather(sxA.at[0], rows0, sg0)
        scatter(dxA.at[0], rows0)
        start_gather(sxB.at[0], rows0, sg0)
        wait_gather(sxA.at[1], rows1, sg1)
        scatter(dxA.at[1], rows1)
        start_gather(sxB.at[1], rows1, sg1)
        start_pair(jnp.minimum(q + 4, last_pair), sxA, dxA, siA)
        wait_gather(sxB.at[0], rows0, sg0)
        scatter(dxB.at[0], rows0)
        wait_pair(sxA, dxA, siA)
        start_gather(sxA.at[0], rows0, sg0)
        wait_gather(sxB.at[1], rows1, sg1)
        scatter(dxB.at[1], rows1)
        start_gather(sxA.at[1], rows1, sg1)
        start_pair(jnp.minimum(q + 6, last_pair), sxB, dxB, siB)
        return carry

    lax.fori_loop(0, NCHUNK // 4, quad, 0)
    wait_gather(sxA.at[0], rows0, sg0)
    wait_gather(sxA.at[1], rows1, sg1)
    wait_pair(sxB, dxB, siB)
    plsc.subcore_barrier()

    for j in range(RPT // WB):
        r0 = s * RPT + j * WB
        pltpu.sync_copy(acc.at[pl.ds(r0, WB)], zsrc)
        pltpu.sync_copy(zsrc, out_hbm.at[c, pl.ds(r0, WB)])


def _add_body(p_ref, o_ref):
    o_ref[...] = p_ref[0] + p_ref[1]


def _combine(partials):
    blk = 1000
    grid = (N_NODES // blk,)
    return pl.pallas_call(
        _add_body,
        grid=grid,
        in_specs=[pl.BlockSpec((NC, blk, EMB), lambda i: (0, i, 0))],
        out_specs=pl.BlockSpec((blk, EMB), lambda i: (i, 0)),
        out_shape=jax.ShapeDtypeStruct((N_NODES, EMB), jnp.float32),
    )(partials)


def kernel(X, edge_index, W1, b1, W2, b2):
    h = _mlp(X, W1, b1.reshape(1, EMB), W2, b2.reshape(1, EMB))
    idx = edge_index.astype(jnp.int32).reshape(2, NW, NCHUNK, K)
    partials = _scatter_kernel(h, idx)
    return _combine(partials)


# speedup vs baseline: 1.4108x; 1.0279x over previous
import functools

import jax
import jax.numpy as jnp
from jax import lax
from jax.experimental import pallas as pl
from jax.experimental.pallas import tpu as pltpu
from jax.experimental.pallas import tpu_sc as plsc

N_NODES = 10000
N_EDGES = 320000
EMB = 128

NC = 2
NS = 16
NW = NC * NS

EPT = N_EDGES // NW
K = 100
NCHUNK = EPT // K
WB = 80
N_PAD = 10240
RPT = N_PAD // NS


def _mlp_body(x_ref, w1_ref, b1_ref, w2_ref, b2_ref, o_ref):
    h = jnp.dot(x_ref[...], w1_ref[...], preferred_element_type=jnp.float32)
    h = jnp.maximum(h + b1_ref[...], 0.0)
    h = jnp.dot(h, w2_ref[...], preferred_element_type=jnp.float32)
    o_ref[...] = jnp.maximum(h + b2_ref[...], 0.0)


def _mlp(X, W1, b1, W2, b2):
    blk = 2000
    grid = (N_NODES // blk,)
    return pl.pallas_call(
        _mlp_body,
        grid=grid,
        in_specs=[
            pl.BlockSpec((blk, EMB), lambda i: (i, 0)),
            pl.BlockSpec((EMB, EMB), lambda i: (0, 0)),
            pl.BlockSpec((1, EMB), lambda i: (0, 0)),
            pl.BlockSpec((EMB, EMB), lambda i: (0, 0)),
            pl.BlockSpec((1, EMB), lambda i: (0, 0)),
        ],
        out_specs=pl.BlockSpec((blk, EMB), lambda i: (i, 0)),
        out_shape=jax.ShapeDtypeStruct((N_NODES, EMB), jnp.float32),
    )(X, W1, b1, W2, b2)


@functools.partial(
    pl.kernel,
    mesh=plsc.VectorSubcoreMesh(core_axis_name="c", subcore_axis_name="s"),
    out_type=jax.ShapeDtypeStruct((NC, N_PAD, EMB), jnp.float32),
    scratch_types=[
        pltpu.VMEM((2, K), jnp.int32),
        pltpu.VMEM((2, K), jnp.int32),
        pltpu.VMEM((2, K), jnp.int32),
        pltpu.VMEM((2, K), jnp.int32),
        pltpu.VMEM((K, EMB), jnp.float32),
        pltpu.VMEM((K, EMB), jnp.float32),
        pltpu.VMEM_SHARED((N_PAD, EMB), jnp.float32),
        pltpu.SemaphoreType.DMA,
        pltpu.SemaphoreType.DMA,
        pltpu.SemaphoreType.DMA,
        pltpu.SemaphoreType.DMA,
    ],
)
def _scatter_kernel(h_hbm, idx_hbm, out_hbm,
                    sxA, dxA, sxB, dxB, rows0, rows1, acc, siA, siB, sg0, sg1):
    c = lax.axis_index("c")
    s = lax.axis_index("s")
    wid = c * NS + s

    zero16 = jnp.zeros((16,), jnp.float32)

    def zero_row(r, carry):
        for j in range(EMB // 16):
            rows0[r, pl.ds(j * 16, 16)] = zero16
        return carry

    lax.fori_loop(0, WB, zero_row, 0)
    zsrc = rows0.at[pl.ds(0, WB)]
    for j in range(RPT // WB):
        pltpu.async_copy(zsrc, acc.at[pl.ds(s * RPT + j * WB, WB)], sg0)
    for j in range(RPT // WB):
        pltpu.make_async_copy(zsrc, acc.at[pl.ds(s * RPT + j * WB, WB)],
                              sg0).wait()
    plsc.subcore_barrier()

    last_pair = NCHUNK - 2

    def start_pair(i, sx, dx, sem):
        pltpu.async_copy(idx_hbm.at[0, wid, pl.ds(i, 2)], sx, sem)
        pltpu.async_copy(idx_hbm.at[1, wid, pl.ds(i, 2)], dx, sem)

    def wait_pair(sx, dx, sem):
        pltpu.make_async_copy(idx_hbm.at[0, wid, pl.ds(0, 2)], sx, sem).wait()
        pltpu.make_async_copy(idx_hbm.at[1, wid, pl.ds(0, 2)], dx, sem).wait()

    def start_gather(isrc, buf, sem):
        pltpu.async_copy(h_hbm.at[isrc], buf, sem)

    def wait_gather(isrc, buf, sem):
        pltpu.make_async_copy(h_hbm.at[isrc], buf, sem).wait()

    def scatter(idst, buf):
        pltpu.sync_copy(buf, acc.at[idst], add=True)

    start_pair(0, sxA, dxA, siA)
    start_pair(2, sxB, dxB, siB)
    wait_pair(sxA, dxA, siA)
    start_gather(sxA.at[0], rows0, sg0)
    start_gather(sxA.at[1], rows1, sg1)

    def quad(g, carry):
        q = 4 * g
        wait_pair(sxB, dxB, siB)
        wait_gather(sxA.at[0], rows0, sg0)
        scatter(dxA.at[0], rows0)
        start_gather(sxB.at[0], rows0, sg0)
        wait_gather(sxA.at[1], rows1, sg1)
        scatter(dxA.at[1], rows1)
        start_gather(sxB.at[1], rows1, sg1)
        start_pair(jnp.minimum(q + 4, last_pair), sxA, dxA, siA)
        wait_gather(sxB.at[0], rows0, sg0)
        scatter(dxB.at[0], rows0)
        wait_pair(sxA, dxA, siA)
        start_gather(sxA.at[0], rows0, sg0)
        wait_gather(sxB.at[1], rows1, sg1)
        scatter(dxB.at[1], rows1)
        start_gather(sxA.at[1], rows1, sg1)
        start_pair(jnp.minimum(q + 6, last_pair), sxB, dxB, siB)
        return carry

    lax.fori_loop(0, NCHUNK // 4, quad, 0)
    wait_gather(sxA.at[0], rows0, sg0)
    wait_gather(sxA.at[1], rows1, sg1)
    wait_pair(sxB, dxB, siB)
    plsc.subcore_barrier()

    pltpu.sync_copy(acc.at[pl.ds(s * RPT, RPT)], out_hbm.at[c, pl.ds(s * RPT, RPT)])


def _add_body(p_ref, o_ref):
    o_ref[...] = p_ref[0] + p_ref[1]


def _combine(partials):
    blk = 1000
    grid = (N_NODES // blk,)
    return pl.pallas_call(
        _add_body,
        grid=grid,
        in_specs=[pl.BlockSpec((NC, blk, EMB), lambda i: (0, i, 0))],
        out_specs=pl.BlockSpec((blk, EMB), lambda i: (i, 0)),
        out_shape=jax.ShapeDtypeStruct((N_NODES, EMB), jnp.float32),
    )(partials)


def kernel(X, edge_index, W1, b1, W2, b2):
    h = _mlp(X, W1, b1.reshape(1, EMB), W2, b2.reshape(1, EMB))
    idx = edge_index.astype(jnp.int32).reshape(2, NW, NCHUNK, K)
    partials = _scatter_kernel(h, idx)
    return _combine(partials)
